# baseline (speedup 1.0000x reference)
import jax
import jax.numpy as jnp
from jax.experimental import pallas as pl
from jax.experimental.pallas import tpu as pltpu

EMBED_DIM = 64
BLOCK_COLS = 128
NUM_NODES_TOTAL = 1000000


def _body(idx_hbm, w_hbm, o_ref, idx_s, wv, sems):
    pltpu.make_async_copy(idx_hbm, idx_s, sems.at[0]).start()
    pltpu.make_async_copy(idx_hbm, idx_s, sems.at[0]).wait()
    idx = idx_s[0]
    base = pl.multiple_of((idx // BLOCK_COLS) * BLOCK_COLS, BLOCK_COLS)
    cols = jnp.minimum(NUM_NODES_TOTAL - base, BLOCK_COLS)
    pltpu.make_async_copy(
        w_hbm.at[:, pl.ds(base, BLOCK_COLS)], wv, sems.at[1]
    ).start()
    pltpu.make_async_copy(
        w_hbm.at[:, pl.ds(base, BLOCK_COLS)], wv, sems.at[1]
    ).wait()
    c = idx - base
    col = jax.lax.broadcasted_iota(jnp.int32, (EMBED_DIM, BLOCK_COLS), 1)
    wcol = jnp.where(col == c, wv[...], 0.0)
    ones = jnp.ones((1, BLOCK_COLS), dtype=jnp.float32)
    o_ref[...] = jax.lax.dot_general(
        ones, wcol, (((1,), (1,)), ((), ())),
        preferred_element_type=jnp.float32,
    )
    del cols


def kernel(node_id, W):
    idx = jnp.asarray(node_id, jnp.int32).reshape((1,))
    Wt = jnp.swapaxes(W, 0, 1)
    return pl.pallas_call(
        _body,
        in_specs=[
            pl.BlockSpec(memory_space=pl.ANY),
            pl.BlockSpec(memory_space=pl.ANY),
        ],
        out_specs=pl.BlockSpec(memory_space=pltpu.VMEM),
        out_shape=jax.ShapeDtypeStruct((1, EMBED_DIM), jnp.float32),
        scratch_shapes=[
            pltpu.SMEM((1,), jnp.int32),
            pltpu.VMEM((EMBED_DIM, BLOCK_COLS), jnp.float32),
            pltpu.SemaphoreType.DMA((2,)),
        ],
    )(idx, Wt)


# speedup vs baseline: 1.2894x; 1.2894x over previous
import jax
import jax.numpy as jnp
from jax.experimental import pallas as pl
from jax.experimental.pallas import tpu as pltpu

EMBED_DIM = 64
BLOCK_COLS = 128


def _lookup_body(idx_ref, w_ref, o_ref):
    c = idx_ref[0] % BLOCK_COLS
    col = jax.lax.broadcasted_iota(jnp.int32, (EMBED_DIM, BLOCK_COLS), 1)
    wcol = jnp.where(col == c, w_ref[...], 0.0)
    o_ref[...] = jnp.sum(wcol, axis=1)[None, :]


def kernel(node_id, W):
    idx = jnp.asarray(node_id, jnp.int32).reshape((1,))
    Wt = jnp.swapaxes(W, 0, 1)
    grid_spec = pltpu.PrefetchScalarGridSpec(
        num_scalar_prefetch=1,
        grid=(1,),
        in_specs=[
            pl.BlockSpec(
                (EMBED_DIM, BLOCK_COLS),
                lambda i, idx_ref: (0, idx_ref[0] // BLOCK_COLS),
            ),
        ],
        out_specs=pl.BlockSpec((1, EMBED_DIM), lambda i, idx_ref: (0, 0)),
    )
    return pl.pallas_call(
        _lookup_body,
        grid_spec=grid_spec,
        out_shape=jax.ShapeDtypeStruct((1, EMBED_DIM), jnp.float32),
    )(idx, Wt)
